# Initial kernel scaffold; baseline (speedup 1.0000x reference)
import functools

import jax
import jax.numpy as jnp
from jax import lax
from jax.experimental import pallas as pl
from jax.experimental.pallas import tpu as pltpu
from jax.experimental.pallas import tpu_sc as plsc

N = 10000
E = 160000
D_IN = 256
D_H = 512

NC = 2
NS = 16
NW = NC * NS

NPAD = 10240
ROWS_PT = NPAD // NS
KB = 128
EP = 163840
EPW = EP // NW
NB_E = EPW // KB
CH = 4
DC = 128
BR = 400
NBLK = N // BR

_mesh = plsc.VectorSubcoreMesh(
    core_axis_name="c", subcore_axis_name="s", num_cores=NC, num_subcores=NS)


def _deg_body(dst_hbm, out_hbm, idx_v, ones_v, zeros_v, acc):
    c = lax.axis_index("c")
    s = lax.axis_index("s")
    wid = c * NS + s

    def _init(i, _):
        ones_v[i, :] = jnp.ones((16,), jnp.float32)
        zeros_v[i, :] = jnp.zeros((16,), jnp.float32)
        return 0

    lax.fori_loop(0, KB, _init, 0)
    for k in range(ROWS_PT // KB):
        pltpu.sync_copy(zeros_v, acc.at[pl.ds(s * ROWS_PT + k * KB, KB)])
    plsc.subcore_barrier()

    base = wid * EPW

    def _step(it, _):
        pltpu.sync_copy(dst_hbm.at[pl.ds(base + it * KB, KB)], idx_v)
        pltpu.sync_copy(ones_v, acc.at[idx_v], add=True)
        return 0

    lax.fori_loop(0, NB_E, _step, 0)
    plsc.subcore_barrier()
    for k in range(ROWS_PT // KB):
        r0 = s * ROWS_PT + k * KB
        pltpu.sync_copy(acc.at[pl.ds(r0, KB)], out_hbm.at[c, pl.ds(r0, KB)])


_deg = functools.partial(
    pl.kernel,
    out_type=jax.ShapeDtypeStruct((NC, NPAD, 16), jnp.float32),
    mesh=_mesh,
    scratch_types=[
        pltpu.VMEM((KB,), jnp.int32),
        pltpu.VMEM((KB, 16), jnp.float32),
        pltpu.VMEM((KB, 16), jnp.float32),
        pltpu.VMEM_SHARED((NPAD, 16), jnp.float32),
    ],
)(_deg_body)


def _segsum_body(y_hbm, src_hbm, dst_hbm, out_hbm,
                 src_v, dst_v, rows_v, zeros_v, acc, sem):
    c = lax.axis_index("c")
    s = lax.axis_index("s")
    wid = c * NS + s
    base = wid * EPW

    def _zrow(i, _):
        for j in range(DC // 16):
            zeros_v[i, pl.ds(j * 16, 16)] = jnp.zeros((16,), jnp.float32)
        return 0

    lax.fori_loop(0, KB, _zrow, 0)

    for ch in range(CH):
        for k in range(ROWS_PT // KB):
            pltpu.sync_copy(zeros_v, acc.at[pl.ds(s * ROWS_PT + k * KB, KB)])
        plsc.subcore_barrier()

        def _step(it, _):
            off = base + it * KB
            pltpu.sync_copy(src_hbm.at[pl.ds(off, KB)], src_v)
            pltpu.sync_copy(dst_hbm.at[pl.ds(off, KB)], dst_v)
            pltpu.async_copy(y_hbm.at[ch].at[src_v], rows_v, sem).wait()
            pltpu.sync_copy(rows_v, acc.at[dst_v], add=True)
            return 0

        lax.fori_loop(0, NB_E, _step, 0)
        plsc.subcore_barrier()
        for k in range(ROWS_PT // KB):
            r0 = s * ROWS_PT + k * KB
            pltpu.sync_copy(acc.at[pl.ds(r0, KB)],
                            out_hbm.at[c, ch, pl.ds(r0, KB)])
        if ch < CH - 1:
            plsc.subcore_barrier()


_segsum = functools.partial(
    pl.kernel,
    out_type=jax.ShapeDtypeStruct((NC, CH, NPAD, DC), jnp.float32),
    mesh=_mesh,
    scratch_types=[
        pltpu.VMEM((KB,), jnp.int32),
        pltpu.VMEM((KB,), jnp.int32),
        pltpu.VMEM((KB, DC), jnp.float32),
        pltpu.VMEM((KB, DC), jnp.float32),
        pltpu.VMEM_SHARED((NPAD, DC), jnp.float32),
        pltpu.SemaphoreType.DMA,
    ],
)(_segsum_body)


def _dinv_of(degp):
    return lax.rsqrt(degp[0, :, 0] + degp[1, :, 0] + 1.0)


def _mm0_body(x_ref, w_ref, degp_ref, out_ref):
    xw = jnp.dot(x_ref[...], w_ref[...], preferred_element_type=jnp.float32)
    y = xw * _dinv_of(degp_ref[...])[:, None]
    out_ref[...] = y.reshape(BR, CH, DC).transpose(1, 0, 2)


def _mm0(x, w, degp):
    return pl.pallas_call(
        _mm0_body,
        grid=(NBLK,),
        in_specs=[
            pl.BlockSpec((BR, D_IN), lambda i: (i, 0)),
            pl.BlockSpec((D_IN, D_H), lambda i: (0, 0)),
            pl.BlockSpec((NC, BR, 16), lambda i: (0, i, 0)),
        ],
        out_specs=pl.BlockSpec((CH, BR, DC), lambda i: (0, i, 0)),
        out_shape=jax.ShapeDtypeStruct((CH, N, DC), jnp.float32),
    )(x, w, degp)


def _combine_body(p_ref, y_ref, degp_ref, b_ref, h_ref, stats_ref, acc_ref):
    i = pl.program_id(0)
    p = p_ref[...]
    seg = p[0] + p[1] + y_ref[...]
    seg = seg.transpose(1, 0, 2).reshape(BR, D_H)
    dinv = _dinv_of(degp_ref[...])
    h = dinv[:, None] * seg + b_ref[...]
    h_ref[...] = h

    @pl.when(i == 0)
    def _():
        acc_ref[...] = jnp.zeros((2, D_H), jnp.float32)

    a = acc_ref[...] + jnp.stack([h.sum(axis=0), (h * h).sum(axis=0)])
    acc_ref[...] = a

    @pl.when(i == NBLK - 1)
    def _():
        mu = a[0] / float(N)
        var = a[1] / float(N) - mu * mu
        rstd = lax.rsqrt(var + 1e-5)
        stats_ref[...] = jnp.concatenate(
            [mu[None], rstd[None], jnp.zeros((6, D_H), jnp.float32)], axis=0)


def _combine(p, y, degp, b):
    return pl.pallas_call(
        _combine_body,
        grid=(NBLK,),
        in_specs=[
            pl.BlockSpec((NC, CH, BR, DC), lambda i: (0, 0, i, 0)),
            pl.BlockSpec((CH, BR, DC), lambda i: (0, i, 0)),
            pl.BlockSpec((NC, BR, 16), lambda i: (0, i, 0)),
            pl.BlockSpec((1, D_H), lambda i: (0, 0)),
        ],
        out_specs=[
            pl.BlockSpec((BR, D_H), lambda i: (i, 0)),
            pl.BlockSpec((8, D_H), lambda i: (0, 0)),
        ],
        out_shape=[
            jax.ShapeDtypeStruct((N, D_H), jnp.float32),
            jax.ShapeDtypeStruct((8, D_H), jnp.float32),
        ],
        scratch_shapes=[pltpu.VMEM((2, D_H), jnp.float32)],
    )(p, y, degp, b)


def _bnmm_body(h_ref, st_ref, g_ref, be_ref, w_ref, degp_ref, out_ref):
    st = st_ref[...]
    z = (h_ref[...] - st[0][None, :]) * st[1][None, :] * g_ref[...] + be_ref[...]
    z = jnp.maximum(z, 0.0)
    y = jnp.dot(z, w_ref[...], preferred_element_type=jnp.float32)
    y = y * _dinv_of(degp_ref[...])[:, None]
    out_ref[...] = y.reshape(BR, CH, DC).transpose(1, 0, 2)


def _bnmm(h, st, g, be, w, degp):
    return pl.pallas_call(
        _bnmm_body,
        grid=(NBLK,),
        in_specs=[
            pl.BlockSpec((BR, D_H), lambda i: (i, 0)),
            pl.BlockSpec((8, D_H), lambda i: (0, 0)),
            pl.BlockSpec((1, D_H), lambda i: (0, 0)),
            pl.BlockSpec((1, D_H), lambda i: (0, 0)),
            pl.BlockSpec((D_H, D_H), lambda i: (0, 0)),
            pl.BlockSpec((NC, BR, 16), lambda i: (0, i, 0)),
        ],
        out_specs=pl.BlockSpec((CH, BR, DC), lambda i: (0, i, 0)),
        out_shape=jax.ShapeDtypeStruct((CH, N, DC), jnp.float32),
    )(h, st, g, be, w, degp)


def _bnapply_body(h_ref, st_ref, g_ref, be_ref, out_ref):
    st = st_ref[...]
    out_ref[...] = ((h_ref[...] - st[0][None, :]) * st[1][None, :]
                    * g_ref[...] + be_ref[...])


def _bnapply(h, st, g, be):
    return pl.pallas_call(
        _bnapply_body,
        grid=(NBLK,),
        in_specs=[
            pl.BlockSpec((BR, D_H), lambda i: (i, 0)),
            pl.BlockSpec((8, D_H), lambda i: (0, 0)),
            pl.BlockSpec((1, D_H), lambda i: (0, 0)),
            pl.BlockSpec((1, D_H), lambda i: (0, 0)),
        ],
        out_specs=pl.BlockSpec((BR, D_H), lambda i: (i, 0)),
        out_shape=jax.ShapeDtypeStruct((N, D_H), jnp.float32),
    )(h, st, g, be)


@jax.jit
def kernel(x, edge_index, W0, b0, g0, be0, W1, b1, g1, be1, W2, b2, g2, be2):
    src = edge_index[0]
    dst = edge_index[1]
    pad = EP - E
    srcp = jnp.concatenate([src, jnp.zeros((pad,), jnp.int32)])
    dstp = jnp.concatenate([dst, jnp.full((pad,), N, jnp.int32)])

    degp = _deg(dstp)

    b0r, g0r, be0r = b0.reshape(1, -1), g0.reshape(1, -1), be0.reshape(1, -1)
    b1r, g1r, be1r = b1.reshape(1, -1), g1.reshape(1, -1), be1.reshape(1, -1)
    b2r, g2r, be2r = b2.reshape(1, -1), g2.reshape(1, -1), be2.reshape(1, -1)

    y0 = _mm0(x, W0, degp)
    p0 = _segsum(y0, srcp, dstp)
    h0, st0 = _combine(p0, y0, degp, b0r)

    y1 = _bnmm(h0, st0, g0r, be0r, W1, degp)
    p1 = _segsum(y1, srcp, dstp)
    h1, st1 = _combine(p1, y1, degp, b1r)

    y2 = _bnmm(h1, st1, g1r, be1r, W2, degp)
    p2 = _segsum(y2, srcp, dstp)
    h2, st2 = _combine(p2, y2, degp, b2r)

    return _bnapply(h2, st2, g2r, be2r)


# speedup vs baseline: 3.0392x; 3.0392x over previous
import functools

import jax
import jax.numpy as jnp
from jax import lax
from jax.experimental import pallas as pl
from jax.experimental.pallas import tpu as pltpu
from jax.experimental.pallas import tpu_sc as plsc

N = 10000
E = 160000
D_IN = 256
D_H = 512

NC = 2
NS = 16
NW = NC * NS

NPAD = 10240
ROWS_PT = NPAD // NS
KB = 128
EP = 163840
EPW = EP // NW
NB_E = EPW // KB
CH = 4
DC = 128
BR = 400
NBLK = N // BR

_mesh = plsc.VectorSubcoreMesh(
    core_axis_name="c", subcore_axis_name="s", num_cores=NC, num_subcores=NS)


def _deg_body(dst_hbm, out_hbm, idx_v, ones_v, zeros_v, acc):
    c = lax.axis_index("c")
    s = lax.axis_index("s")
    wid = c * NS + s

    def _init(i, _):
        for j in range(DC // 16):
            ones_v[i, pl.ds(j * 16, 16)] = jnp.ones((16,), jnp.float32)
            zeros_v[i, pl.ds(j * 16, 16)] = jnp.zeros((16,), jnp.float32)
        return 0

    lax.fori_loop(0, KB, _init, 0)
    for k in range(ROWS_PT // KB):
        pltpu.sync_copy(zeros_v, acc.at[pl.ds(s * ROWS_PT + k * KB, KB)])
    plsc.subcore_barrier()

    base = wid * EPW

    def _step(it, _):
        pltpu.sync_copy(dst_hbm.at[pl.ds(base + it * KB, KB)], idx_v)
        pltpu.sync_copy(ones_v, acc.at[idx_v], add=True)
        return 0

    lax.fori_loop(0, NB_E, _step, 0)
    plsc.subcore_barrier()
    for k in range(ROWS_PT // KB):
        r0 = s * ROWS_PT + k * KB
        pltpu.sync_copy(acc.at[pl.ds(r0, KB)], out_hbm.at[c, pl.ds(r0, KB)])


_deg = functools.partial(
    pl.kernel,
    out_type=jax.ShapeDtypeStruct((NC, NPAD, DC), jnp.float32),
    mesh=_mesh,
    scratch_types=[
        pltpu.VMEM((KB,), jnp.int32),
        pltpu.VMEM((KB, DC), jnp.float32),
        pltpu.VMEM((KB, DC), jnp.float32),
        pltpu.VMEM_SHARED((NPAD, DC), jnp.float32),
    ],
)(_deg_body)


def _segsum_body(y_hbm, src_hbm, dst_hbm, out_hbm,
                 src_v, dst_v, rows_v, zeros_v, acc, sem):
    c = lax.axis_index("c")
    s = lax.axis_index("s")
    wid = c * NS + s
    base = wid * EPW

    def _zrow(i, _):
        for j in range(DC // 16):
            zeros_v[i, pl.ds(j * 16, 16)] = jnp.zeros((16,), jnp.float32)
        return 0

    lax.fori_loop(0, KB, _zrow, 0)

    for ch in range(CH):
        for k in range(ROWS_PT // KB):
            pltpu.sync_copy(zeros_v, acc.at[pl.ds(s * ROWS_PT + k * KB, KB)])
        plsc.subcore_barrier()

        def _step(it, _):
            off = base + it * KB
            pltpu.sync_copy(src_hbm.at[pl.ds(off, KB)], src_v)
            pltpu.sync_copy(dst_hbm.at[pl.ds(off, KB)], dst_v)
            pltpu.async_copy(y_hbm.at[ch].at[src_v], rows_v, sem).wait()
            pltpu.sync_copy(rows_v, acc.at[dst_v], add=True)
            return 0

        lax.fori_loop(0, NB_E, _step, 0)
        plsc.subcore_barrier()
        for k in range(ROWS_PT // KB):
            r0 = s * ROWS_PT + k * KB
            pltpu.sync_copy(acc.at[pl.ds(r0, KB)],
                            out_hbm.at[c, ch, pl.ds(r0, KB)])
        if ch < CH - 1:
            plsc.subcore_barrier()


_segsum = functools.partial(
    pl.kernel,
    out_type=jax.ShapeDtypeStruct((NC, CH, NPAD, DC), jnp.float32),
    mesh=_mesh,
    scratch_types=[
        pltpu.VMEM((KB,), jnp.int32),
        pltpu.VMEM((KB,), jnp.int32),
        pltpu.VMEM((KB, DC), jnp.float32),
        pltpu.VMEM((KB, DC), jnp.float32),
        pltpu.VMEM_SHARED((NPAD, DC), jnp.float32),
        pltpu.SemaphoreType.DMA,
    ],
)(_segsum_body)


def _dinv_of(degp):
    return lax.rsqrt(degp[0, :, 0] + degp[1, :, 0] + 1.0)


def _mm0_body(x_ref, w_ref, degp_ref, out_ref):
    xw = jnp.dot(x_ref[...], w_ref[...], preferred_element_type=jnp.float32)
    y = xw * _dinv_of(degp_ref[...])[:, None]
    out_ref[...] = y.reshape(BR, CH, DC).transpose(1, 0, 2)


def _mm0(x, w, degp):
    return pl.pallas_call(
        _mm0_body,
        grid=(NBLK,),
        in_specs=[
            pl.BlockSpec((BR, D_IN), lambda i: (i, 0)),
            pl.BlockSpec((D_IN, D_H), lambda i: (0, 0)),
            pl.BlockSpec((NC, BR, 16), lambda i: (0, i, 0)),
        ],
        out_specs=pl.BlockSpec((CH, BR, DC), lambda i: (0, i, 0)),
        out_shape=jax.ShapeDtypeStruct((CH, N, DC), jnp.float32),
    )(x, w, degp)


def _combine_body(p_ref, y_ref, degp_ref, b_ref, h_ref, stats_ref, acc_ref):
    i = pl.program_id(0)
    p = p_ref[...]
    seg = p[0] + p[1] + y_ref[...]
    seg = seg.transpose(1, 0, 2).reshape(BR, D_H)
    dinv = _dinv_of(degp_ref[...])
    h = dinv[:, None] * seg + b_ref[...]
    h_ref[...] = h

    @pl.when(i == 0)
    def _():
        acc_ref[...] = jnp.zeros((2, D_H), jnp.float32)

    a = acc_ref[...] + jnp.stack([h.sum(axis=0), (h * h).sum(axis=0)])
    acc_ref[...] = a

    @pl.when(i == NBLK - 1)
    def _():
        mu = a[0] / float(N)
        var = a[1] / float(N) - mu * mu
        rstd = lax.rsqrt(var + 1e-5)
        stats_ref[...] = jnp.concatenate(
            [mu[None], rstd[None], jnp.zeros((6, D_H), jnp.float32)], axis=0)


def _combine(p, y, degp, b):
    return pl.pallas_call(
        _combine_body,
        grid=(NBLK,),
        in_specs=[
            pl.BlockSpec((NC, CH, BR, DC), lambda i: (0, 0, i, 0)),
            pl.BlockSpec((CH, BR, DC), lambda i: (0, i, 0)),
            pl.BlockSpec((NC, BR, 16), lambda i: (0, i, 0)),
            pl.BlockSpec((1, D_H), lambda i: (0, 0)),
        ],
        out_specs=[
            pl.BlockSpec((BR, D_H), lambda i: (i, 0)),
            pl.BlockSpec((8, D_H), lambda i: (0, 0)),
        ],
        out_shape=[
            jax.ShapeDtypeStruct((N, D_H), jnp.float32),
            jax.ShapeDtypeStruct((8, D_H), jnp.float32),
        ],
        scratch_shapes=[pltpu.VMEM((2, D_H), jnp.float32)],
    )(p, y, degp, b)


def _bnmm_body(h_ref, st_ref, g_ref, be_ref, w_ref, degp_ref, out_ref):
    st = st_ref[...]
    z = (h_ref[...] - st[0][None, :]) * st[1][None, :] * g_ref[...] + be_ref[...]
    z = jnp.maximum(z, 0.0)
    y = jnp.dot(z, w_ref[...], preferred_element_type=jnp.float32)
    y = y * _dinv_of(degp_ref[...])[:, None]
    out_ref[...] = y.reshape(BR, CH, DC).transpose(1, 0, 2)


def _bnmm(h, st, g, be, w, degp):
    return pl.pallas_call(
        _bnmm_body,
        grid=(NBLK,),
        in_specs=[
            pl.BlockSpec((BR, D_H), lambda i: (i, 0)),
            pl.BlockSpec((8, D_H), lambda i: (0, 0)),
            pl.BlockSpec((1, D_H), lambda i: (0, 0)),
            pl.BlockSpec((1, D_H), lambda i: (0, 0)),
            pl.BlockSpec((D_H, D_H), lambda i: (0, 0)),
            pl.BlockSpec((NC, BR, 16), lambda i: (0, i, 0)),
        ],
        out_specs=pl.BlockSpec((CH, BR, DC), lambda i: (0, i, 0)),
        out_shape=jax.ShapeDtypeStruct((CH, N, DC), jnp.float32),
    )(h, st, g, be, w, degp)


def _bnapply_body(h_ref, st_ref, g_ref, be_ref, out_ref):
    st = st_ref[...]
    out_ref[...] = ((h_ref[...] - st[0][None, :]) * st[1][None, :]
                    * g_ref[...] + be_ref[...])


def _bnapply(h, st, g, be):
    return pl.pallas_call(
        _bnapply_body,
        grid=(NBLK,),
        in_specs=[
            pl.BlockSpec((BR, D_H), lambda i: (i, 0)),
            pl.BlockSpec((8, D_H), lambda i: (0, 0)),
            pl.BlockSpec((1, D_H), lambda i: (0, 0)),
            pl.BlockSpec((1, D_H), lambda i: (0, 0)),
        ],
        out_specs=pl.BlockSpec((BR, D_H), lambda i: (i, 0)),
        out_shape=jax.ShapeDtypeStruct((N, D_H), jnp.float32),
    )(h, st, g, be)


@jax.jit
def kernel(x, edge_index, W0, b0, g0, be0, W1, b1, g1, be1, W2, b2, g2, be2):
    src = edge_index[0]
    dst = edge_index[1]
    pad = EP - E
    srcp = jnp.concatenate([src, jnp.zeros((pad,), jnp.int32)])
    dstp = jnp.concatenate([dst, jnp.full((pad,), N, jnp.int32)])

    degp = _deg(dstp)[:, :, :16]

    b0r, g0r, be0r = b0.reshape(1, -1), g0.reshape(1, -1), be0.reshape(1, -1)
    b1r, g1r, be1r = b1.reshape(1, -1), g1.reshape(1, -1), be1.reshape(1, -1)
    b2r, g2r, be2r = b2.reshape(1, -1), g2.reshape(1, -1), be2.reshape(1, -1)

    y0 = _mm0(x, W0, degp)
    p0 = _segsum(y0, srcp, dstp)
    h0, st0 = _combine(p0, y0, degp, b0r)

    y1 = _bnmm(h0, st0, g0r, be0r, W1, degp)
    p1 = _segsum(y1, srcp, dstp)
    h1, st1 = _combine(p1, y1, degp, b1r)

    y2 = _bnmm(h1, st1, g1r, be1r, W2, degp)
    p2 = _segsum(y2, srcp, dstp)
    h2, st2 = _combine(p2, y2, degp, b2r)

    return _bnapply(h2, st2, g2r, be2r)


# speedup vs baseline: 3.2225x; 1.0603x over previous
import functools

import jax
import jax.numpy as jnp
from jax import lax
from jax.experimental import pallas as pl
from jax.experimental.pallas import tpu as pltpu
from jax.experimental.pallas import tpu_sc as plsc

N = 10000
E = 160000
D_IN = 256
D_H = 512

NC = 2
NS = 16
NW = NC * NS

NPAD = 10240
ROWS_PT = NPAD // NS
KB = 128
EP = 163840
EPW = EP // NW
NB_E = EPW // KB
CH = 4
DC = 128
BR = 400
NBLK = N // BR

_mesh = plsc.VectorSubcoreMesh(
    core_axis_name="c", subcore_axis_name="s", num_cores=NC, num_subcores=NS)


NQ = 4
QROWS = NPAD // NQ
DACC = QROWS + KB
DROWS_PT = DACC // NS
QROWS_PT = QROWS // NS


def _deg_body(dst_hbm, out_hbm, dst_i, i2a, i2b, ones_v, zeros_v, acc,
              ssA, ssB):
    c = lax.axis_index("c")
    s = lax.axis_index("s")
    wid = c * NS + s

    def _init(i, _):
        for j in range(DC // 16):
            ones_v[i, pl.ds(j * 16, 16)] = jnp.ones((16,), jnp.float32)
            zeros_v[i, pl.ds(j * 16, 16)] = jnp.zeros((16,), jnp.float32)
        return 0

    lax.fori_loop(0, KB, _init, 0)
    pltpu.sync_copy(dst_hbm.at[wid], dst_i)

    def _drain(sem):
        pltpu.make_async_copy(out_hbm.at[0, pl.ds(0, KB)], ones_v, sem).wait()

    for q in range(NQ):
        lo = q * QROWS
        pltpu.sync_copy(zeros_v, acc.at[pl.ds(s * DROWS_PT, KB)])
        pltpu.sync_copy(zeros_v.at[pl.ds(0, DROWS_PT - KB)],
                        acc.at[pl.ds(s * DROWS_PT + KB, DROWS_PT - KB)])
        plsc.subcore_barrier()

        def _remap(i, buf):
            for j in range(KB // 16):
                v = dst_i[i, pl.ds(j * 16, 16)]
                ok = (v >= lo) & (v < lo + QROWS)
                buf[pl.ds(j * 16, 16)] = jnp.where(ok, v - lo, DACC - 1)

        def _body(j, _):
            @pl.when(j > 0)
            def _():
                _drain(ssA)

            _remap(2 * j, i2a)
            pltpu.async_copy(ones_v, acc.at[i2a], ssA, add=True)

            @pl.when(j > 0)
            def _():
                _drain(ssB)

            _remap(2 * j + 1, i2b)
            pltpu.async_copy(ones_v, acc.at[i2b], ssB, add=True)
            return 0

        lax.fori_loop(0, NB_E // 2, _body, 0)
        _drain(ssA)
        _drain(ssB)
        plsc.subcore_barrier()
        r0 = s * QROWS_PT
        pltpu.sync_copy(acc.at[pl.ds(r0, KB)],
                        out_hbm.at[c, pl.ds(lo + r0, KB)])
        pltpu.sync_copy(acc.at[pl.ds(r0 + KB, QROWS_PT - KB)],
                        out_hbm.at[c, pl.ds(lo + r0 + KB, QROWS_PT - KB)])
        if q < NQ - 1:
            plsc.subcore_barrier()


_deg = functools.partial(
    pl.kernel,
    out_type=jax.ShapeDtypeStruct((NC, NPAD, DC), jnp.float32),
    mesh=_mesh,
    scratch_types=[
        pltpu.VMEM((NB_E, KB), jnp.int32),
        pltpu.VMEM((KB,), jnp.int32),
        pltpu.VMEM((KB,), jnp.int32),
        pltpu.VMEM((KB, DC), jnp.float32),
        pltpu.VMEM((KB, DC), jnp.float32),
        pltpu.VMEM_SHARED((DACC, DC), jnp.float32),
        pltpu.SemaphoreType.DMA,
        pltpu.SemaphoreType.DMA,
    ],
)(_deg_body)


def _segsum_body(y_hbm, src_hbm, dst_hbm, out_hbm,
                 src_i, dst_i, ra0, ra1, rb0, rb1, zeros_v, acc,
                 gsA, gsB, ssA, ssB):
    c = lax.axis_index("c")
    s = lax.axis_index("s")
    wid = c * NS + s

    def _zrow(i, _):
        for j in range(DC // 16):
            zeros_v[i, pl.ds(j * 16, 16)] = jnp.zeros((16,), jnp.float32)
        return 0

    lax.fori_loop(0, KB, _zrow, 0)
    pltpu.sync_copy(src_hbm.at[wid], src_i)
    pltpu.sync_copy(dst_hbm.at[wid], dst_i)

    for ch in range(CH):
        tab = y_hbm.at[ch]
        for k in range(ROWS_PT // KB):
            pltpu.sync_copy(zeros_v, acc.at[pl.ds(s * ROWS_PT + k * KB, KB)])
        plsc.subcore_barrier()

        def _step(it, _):
            pltpu.async_copy(tab.at[src_i.at[it]], ra0, gsA).wait()
            pltpu.sync_copy(ra0, acc.at[dst_i.at[it]], add=True)
            return 0

        lax.fori_loop(0, NB_E, _step, 0)
        plsc.subcore_barrier()
        for k in range(ROWS_PT // KB):
            r0 = s * ROWS_PT + k * KB
            pltpu.sync_copy(acc.at[pl.ds(r0, KB)],
                            out_hbm.at[c, ch, pl.ds(r0, KB)])
        if ch < CH - 1:
            plsc.subcore_barrier()


_segsum = functools.partial(
    pl.kernel,
    out_type=jax.ShapeDtypeStruct((NC, CH, NPAD, DC), jnp.float32),
    mesh=_mesh,
    scratch_types=[
        pltpu.VMEM((NB_E, KB), jnp.int32),
        pltpu.VMEM((NB_E, KB), jnp.int32),
        pltpu.VMEM((KB, DC), jnp.float32),
        pltpu.VMEM((KB, DC), jnp.float32),
        pltpu.VMEM((KB, DC), jnp.float32),
        pltpu.VMEM((KB, DC), jnp.float32),
        pltpu.VMEM((KB, DC), jnp.float32),
        pltpu.VMEM_SHARED((NPAD, DC), jnp.float32),
        pltpu.SemaphoreType.DMA,
        pltpu.SemaphoreType.DMA,
        pltpu.SemaphoreType.DMA,
        pltpu.SemaphoreType.DMA,
    ],
)(_segsum_body)


def _dinv_of(degp):
    return lax.rsqrt(degp[0, :, 0] + degp[1, :, 0] + 1.0)


def _mm0_body(x_ref, w_ref, degp_ref, out_ref):
    xw = jnp.dot(x_ref[...], w_ref[...], preferred_element_type=jnp.float32)
    y = xw * _dinv_of(degp_ref[...])[:, None]
    out_ref[...] = y.reshape(BR, CH, DC).transpose(1, 0, 2)


def _mm0(x, w, degp):
    return pl.pallas_call(
        _mm0_body,
        grid=(NBLK,),
        in_specs=[
            pl.BlockSpec((BR, D_IN), lambda i: (i, 0)),
            pl.BlockSpec((D_IN, D_H), lambda i: (0, 0)),
            pl.BlockSpec((NC, BR, 16), lambda i: (0, i, 0)),
        ],
        out_specs=pl.BlockSpec((CH, BR, DC), lambda i: (0, i, 0)),
        out_shape=jax.ShapeDtypeStruct((CH, N, DC), jnp.float32),
    )(x, w, degp)


def _combine_body(p_ref, y_ref, degp_ref, b_ref, h_ref, stats_ref, acc_ref):
    i = pl.program_id(0)
    p = p_ref[...]
    seg = p[0] + p[1] + y_ref[...]
    seg = seg.transpose(1, 0, 2).reshape(BR, D_H)
    dinv = _dinv_of(degp_ref[...])
    h = dinv[:, None] * seg + b_ref[...]
    h_ref[...] = h

    @pl.when(i == 0)
    def _():
        acc_ref[...] = jnp.zeros((2, D_H), jnp.float32)

    a = acc_ref[...] + jnp.stack([h.sum(axis=0), (h * h).sum(axis=0)])
    acc_ref[...] = a

    @pl.when(i == NBLK - 1)
    def _():
        mu = a[0] / float(N)
        var = a[1] / float(N) - mu * mu
        rstd = lax.rsqrt(var + 1e-5)
        stats_ref[...] = jnp.concatenate(
            [mu[None], rstd[None], jnp.zeros((6, D_H), jnp.float32)], axis=0)


def _combine(p, y, degp, b):
    return pl.pallas_call(
        _combine_body,
        grid=(NBLK,),
        in_specs=[
            pl.BlockSpec((NC, CH, BR, DC), lambda i: (0, 0, i, 0)),
            pl.BlockSpec((CH, BR, DC), lambda i: (0, i, 0)),
            pl.BlockSpec((NC, BR, 16), lambda i: (0, i, 0)),
            pl.BlockSpec((1, D_H), lambda i: (0, 0)),
        ],
        out_specs=[
            pl.BlockSpec((BR, D_H), lambda i: (i, 0)),
            pl.BlockSpec((8, D_H), lambda i: (0, 0)),
        ],
        out_shape=[
            jax.ShapeDtypeStruct((N, D_H), jnp.float32),
            jax.ShapeDtypeStruct((8, D_H), jnp.float32),
        ],
        scratch_shapes=[pltpu.VMEM((2, D_H), jnp.float32)],
    )(p, y, degp, b)


def _bnmm_body(h_ref, st_ref, g_ref, be_ref, w_ref, degp_ref, out_ref):
    st = st_ref[...]
    z = (h_ref[...] - st[0][None, :]) * st[1][None, :] * g_ref[...] + be_ref[...]
    z = jnp.maximum(z, 0.0)
    y = jnp.dot(z, w_ref[...], preferred_element_type=jnp.float32)
    y = y * _dinv_of(degp_ref[...])[:, None]
    out_ref[...] = y.reshape(BR, CH, DC).transpose(1, 0, 2)


def _bnmm(h, st, g, be, w, degp):
    return pl.pallas_call(
        _bnmm_body,
        grid=(NBLK,),
        in_specs=[
            pl.BlockSpec((BR, D_H), lambda i: (i, 0)),
            pl.BlockSpec((8, D_H), lambda i: (0, 0)),
            pl.BlockSpec((1, D_H), lambda i: (0, 0)),
            pl.BlockSpec((1, D_H), lambda i: (0, 0)),
            pl.BlockSpec((D_H, D_H), lambda i: (0, 0)),
            pl.BlockSpec((NC, BR, 16), lambda i: (0, i, 0)),
        ],
        out_specs=pl.BlockSpec((CH, BR, DC), lambda i: (0, i, 0)),
        out_shape=jax.ShapeDtypeStruct((CH, N, DC), jnp.float32),
    )(h, st, g, be, w, degp)


def _bnapply_body(h_ref, st_ref, g_ref, be_ref, out_ref):
    st = st_ref[...]
    out_ref[...] = ((h_ref[...] - st[0][None, :]) * st[1][None, :]
                    * g_ref[...] + be_ref[...])


def _bnapply(h, st, g, be):
    return pl.pallas_call(
        _bnapply_body,
        grid=(NBLK,),
        in_specs=[
            pl.BlockSpec((BR, D_H), lambda i: (i, 0)),
            pl.BlockSpec((8, D_H), lambda i: (0, 0)),
            pl.BlockSpec((1, D_H), lambda i: (0, 0)),
            pl.BlockSpec((1, D_H), lambda i: (0, 0)),
        ],
        out_specs=pl.BlockSpec((BR, D_H), lambda i: (i, 0)),
        out_shape=jax.ShapeDtypeStruct((N, D_H), jnp.float32),
    )(h, st, g, be)


@jax.jit
def kernel(x, edge_index, W0, b0, g0, be0, W1, b1, g1, be1, W2, b2, g2, be2):
    src = edge_index[0]
    dst = edge_index[1]
    pad = EP - E
    srcp = jnp.concatenate([src, jnp.zeros((pad,), jnp.int32)])
    srcp = srcp.reshape(NW, NB_E, KB)
    dstp = jnp.concatenate([dst, jnp.full((pad,), N, jnp.int32)])
    dstp = dstp.reshape(NW, NB_E, KB)

    degp = _deg(dstp)[:, :, :16]

    b0r, g0r, be0r = b0.reshape(1, -1), g0.reshape(1, -1), be0.reshape(1, -1)
    b1r, g1r, be1r = b1.reshape(1, -1), g1.reshape(1, -1), be1.reshape(1, -1)
    b2r, g2r, be2r = b2.reshape(1, -1), g2.reshape(1, -1), be2.reshape(1, -1)

    y0 = _mm0(x, W0, degp)
    p0 = _segsum(y0, srcp, dstp)
    h0, st0 = _combine(p0, y0, degp, b0r)

    y1 = _bnmm(h0, st0, g0r, be0r, W1, degp)
    p1 = _segsum(y1, srcp, dstp)
    h1, st1 = _combine(p1, y1, degp, b1r)

    y2 = _bnmm(h1, st1, g1r, be1r, W2, degp)
    p2 = _segsum(y2, srcp, dstp)
    h2, st2 = _combine(p2, y2, degp, b2r)

    return _bnapply(h2, st2, g2r, be2r)


# speedup vs baseline: 3.3532x; 1.0405x over previous
import functools

import jax
import jax.numpy as jnp
from jax import lax
from jax.experimental import pallas as pl
from jax.experimental.pallas import tpu as pltpu
from jax.experimental.pallas import tpu_sc as plsc

N = 10000
E = 160000
D_IN = 256
D_H = 512

NC = 2
NS = 16
NW = NC * NS

NPAD = 10240
ROWS_PT = NPAD // NS
KB = 128
EP = 163840
EPW = EP // NW
NB_E = EPW // KB
CH = 4
DC = 128
BR = 400
NBLK = N // BR

_mesh = plsc.VectorSubcoreMesh(
    core_axis_name="c", subcore_axis_name="s", num_cores=NC, num_subcores=NS)


def _deg_body(dst_hbm, out_hbm, dst_i, ones_v, zeros_v, acc, gsA):
    c = lax.axis_index("c")
    s = lax.axis_index("s")
    wid = c * NS + s

    def _init(i, _):
        for j in range(DC // 16):
            ones_v[i, pl.ds(j * 16, 16)] = jnp.ones((16,), jnp.float32)
            zeros_v[i, pl.ds(j * 16, 16)] = jnp.zeros((16,), jnp.float32)
        return 0

    lax.fori_loop(0, KB, _init, 0)
    pltpu.sync_copy(dst_hbm.at[wid], dst_i)
    for k in range(ROWS_PT // KB):
        pltpu.sync_copy(zeros_v, acc.at[pl.ds(s * ROWS_PT + k * KB, KB)])
    plsc.subcore_barrier()

    def _step(it, _):
        pltpu.sync_copy(ones_v, acc.at[dst_i.at[it]], add=True)
        return 0

    lax.fori_loop(0, NB_E, _step, 0)
    plsc.subcore_barrier()
    for k in range(ROWS_PT // KB):
        r0 = s * ROWS_PT + k * KB
        pltpu.sync_copy(acc.at[pl.ds(r0, KB)], out_hbm.at[c, pl.ds(r0, KB)])


_deg = functools.partial(
    pl.kernel,
    out_type=jax.ShapeDtypeStruct((NC, NPAD, DC), jnp.float32),
    mesh=_mesh,
    scratch_types=[
        pltpu.VMEM((NB_E, KB), jnp.int32),
        pltpu.VMEM((KB, DC), jnp.float32),
        pltpu.VMEM((KB, DC), jnp.float32),
        pltpu.VMEM_SHARED((NPAD, DC), jnp.float32),
        pltpu.SemaphoreType.DMA,
    ],
)(_deg_body)


def _segsum_body(y_hbm, src_hbm, dst_hbm, out_hbm,
                 src_i, dst_i, ra0, ra1, rb0, rb1, zeros_v, acc,
                 gsA, gsB, ssA, ssB):
    c = lax.axis_index("c")
    s = lax.axis_index("s")
    wid = c * NS + s

    def _zrow(i, _):
        for j in range(DC // 16):
            zeros_v[i, pl.ds(j * 16, 16)] = jnp.zeros((16,), jnp.float32)
        return 0

    lax.fori_loop(0, KB, _zrow, 0)
    pltpu.sync_copy(src_hbm.at[wid], src_i)
    pltpu.sync_copy(dst_hbm.at[wid], dst_i)

    for ch in range(CH):
        tab = y_hbm.at[ch]
        for k in range(ROWS_PT // KB):
            pltpu.sync_copy(zeros_v, acc.at[pl.ds(s * ROWS_PT + k * KB, KB)])
        plsc.subcore_barrier()

        def _step(it, _):
            pltpu.async_copy(tab.at[src_i.at[it]], ra0, gsA).wait()
            pltpu.sync_copy(ra0, acc.at[dst_i.at[it]], add=True)
            return 0

        lax.fori_loop(0, NB_E, _step, 0)
        plsc.subcore_barrier()
        for k in range(ROWS_PT // KB):
            r0 = s * ROWS_PT + k * KB
            pltpu.sync_copy(acc.at[pl.ds(r0, KB)],
                            out_hbm.at[c, ch, pl.ds(r0, KB)])
        if ch < CH - 1:
            plsc.subcore_barrier()


_segsum = functools.partial(
    pl.kernel,
    out_type=jax.ShapeDtypeStruct((NC, CH, NPAD, DC), jnp.float32),
    mesh=_mesh,
    scratch_types=[
        pltpu.VMEM((NB_E, KB), jnp.int32),
        pltpu.VMEM((NB_E, KB), jnp.int32),
        pltpu.VMEM((KB, DC), jnp.float32),
        pltpu.VMEM((KB, DC), jnp.float32),
        pltpu.VMEM((KB, DC), jnp.float32),
        pltpu.VMEM((KB, DC), jnp.float32),
        pltpu.VMEM((KB, DC), jnp.float32),
        pltpu.VMEM_SHARED((NPAD, DC), jnp.float32),
        pltpu.SemaphoreType.DMA,
        pltpu.SemaphoreType.DMA,
        pltpu.SemaphoreType.DMA,
        pltpu.SemaphoreType.DMA,
    ],
)(_segsum_body)


def _dinv_of(degp):
    return lax.rsqrt(degp[0, :, 0] + degp[1, :, 0] + 1.0)


def _mm0_body(x_ref, w_ref, degp_ref, out_ref):
    xw = jnp.dot(x_ref[...], w_ref[...], preferred_element_type=jnp.float32)
    y = xw * _dinv_of(degp_ref[...])[:, None]
    out_ref[...] = y.reshape(BR, CH, DC).transpose(1, 0, 2)


def _mm0(x, w, degp):
    return pl.pallas_call(
        _mm0_body,
        grid=(NBLK,),
        in_specs=[
            pl.BlockSpec((BR, D_IN), lambda i: (i, 0)),
            pl.BlockSpec((D_IN, D_H), lambda i: (0, 0)),
            pl.BlockSpec((NC, BR, 16), lambda i: (0, i, 0)),
        ],
        out_specs=pl.BlockSpec((CH, BR, DC), lambda i: (0, i, 0)),
        out_shape=jax.ShapeDtypeStruct((CH, N, DC), jnp.float32),
    )(x, w, degp)


def _combine_body(p_ref, y_ref, degp_ref, b_ref, h_ref, stats_ref, acc_ref):
    i = pl.program_id(0)
    p = p_ref[...]
    seg = p[0] + p[1] + y_ref[...]
    seg = seg.transpose(1, 0, 2).reshape(BR, D_H)
    dinv = _dinv_of(degp_ref[...])
    h = dinv[:, None] * seg + b_ref[...]
    h_ref[...] = h

    @pl.when(i == 0)
    def _():
        acc_ref[...] = jnp.zeros((2, D_H), jnp.float32)

    a = acc_ref[...] + jnp.stack([h.sum(axis=0), (h * h).sum(axis=0)])
    acc_ref[...] = a

    @pl.when(i == NBLK - 1)
    def _():
        mu = a[0] / float(N)
        var = a[1] / float(N) - mu * mu
        rstd = lax.rsqrt(var + 1e-5)
        stats_ref[...] = jnp.concatenate(
            [mu[None], rstd[None], jnp.zeros((6, D_H), jnp.float32)], axis=0)


def _combine(p, y, degp, b):
    return pl.pallas_call(
        _combine_body,
        grid=(NBLK,),
        in_specs=[
            pl.BlockSpec((NC, CH, BR, DC), lambda i: (0, 0, i, 0)),
            pl.BlockSpec((CH, BR, DC), lambda i: (0, i, 0)),
            pl.BlockSpec((NC, BR, 16), lambda i: (0, i, 0)),
            pl.BlockSpec((1, D_H), lambda i: (0, 0)),
        ],
        out_specs=[
            pl.BlockSpec((BR, D_H), lambda i: (i, 0)),
            pl.BlockSpec((8, D_H), lambda i: (0, 0)),
        ],
        out_shape=[
            jax.ShapeDtypeStruct((N, D_H), jnp.float32),
            jax.ShapeDtypeStruct((8, D_H), jnp.float32),
        ],
        scratch_shapes=[pltpu.VMEM((2, D_H), jnp.float32)],
    )(p, y, degp, b)


def _bnmm_body(h_ref, st_ref, g_ref, be_ref, w_ref, degp_ref, out_ref):
    st = st_ref[...]
    z = (h_ref[...] - st[0][None, :]) * st[1][None, :] * g_ref[...] + be_ref[...]
    z = jnp.maximum(z, 0.0)
    y = jnp.dot(z, w_ref[...], preferred_element_type=jnp.float32)
    y = y * _dinv_of(degp_ref[...])[:, None]
    out_ref[...] = y.reshape(BR, CH, DC).transpose(1, 0, 2)


def _bnmm(h, st, g, be, w, degp):
    return pl.pallas_call(
        _bnmm_body,
        grid=(NBLK,),
        in_specs=[
            pl.BlockSpec((BR, D_H), lambda i: (i, 0)),
            pl.BlockSpec((8, D_H), lambda i: (0, 0)),
            pl.BlockSpec((1, D_H), lambda i: (0, 0)),
            pl.BlockSpec((1, D_H), lambda i: (0, 0)),
            pl.BlockSpec((D_H, D_H), lambda i: (0, 0)),
            pl.BlockSpec((NC, BR, 16), lambda i: (0, i, 0)),
        ],
        out_specs=pl.BlockSpec((CH, BR, DC), lambda i: (0, i, 0)),
        out_shape=jax.ShapeDtypeStruct((CH, N, DC), jnp.float32),
    )(h, st, g, be, w, degp)


def _bnapply_body(h_ref, st_ref, g_ref, be_ref, out_ref):
    st = st_ref[...]
    out_ref[...] = ((h_ref[...] - st[0][None, :]) * st[1][None, :]
                    * g_ref[...] + be_ref[...])


def _bnapply(h, st, g, be):
    return pl.pallas_call(
        _bnapply_body,
        grid=(NBLK,),
        in_specs=[
            pl.BlockSpec((BR, D_H), lambda i: (i, 0)),
            pl.BlockSpec((8, D_H), lambda i: (0, 0)),
            pl.BlockSpec((1, D_H), lambda i: (0, 0)),
            pl.BlockSpec((1, D_H), lambda i: (0, 0)),
        ],
        out_specs=pl.BlockSpec((BR, D_H), lambda i: (i, 0)),
        out_shape=jax.ShapeDtypeStruct((N, D_H), jnp.float32),
    )(h, st, g, be)


@jax.jit
def kernel(x, edge_index, W0, b0, g0, be0, W1, b1, g1, be1, W2, b2, g2, be2):
    src = edge_index[0]
    dst = edge_index[1]
    pad = EP - E
    srcp = jnp.concatenate([src, jnp.zeros((pad,), jnp.int32)])
    srcp = srcp.reshape(NW, NB_E, KB)
    dstp = jnp.concatenate([dst, jnp.full((pad,), N, jnp.int32)])
    dstp = dstp.reshape(NW, NB_E, KB)

    degp = _deg(dstp)[:, :, :16]

    b0r, g0r, be0r = b0.reshape(1, -1), g0.reshape(1, -1), be0.reshape(1, -1)
    b1r, g1r, be1r = b1.reshape(1, -1), g1.reshape(1, -1), be1.reshape(1, -1)
    b2r, g2r, be2r = b2.reshape(1, -1), g2.reshape(1, -1), be2.reshape(1, -1)

    y0 = _mm0(x, W0, degp)
    p0 = _segsum(y0, srcp, dstp)
    h0, st0 = _combine(p0, y0, degp, b0r)

    y1 = _bnmm(h0, st0, g0r, be0r, W1, degp)
    p1 = _segsum(y1, srcp, dstp)
    h1, st1 = _combine(p1, y1, degp, b1r)

    y2 = _bnmm(h1, st1, g1r, be1r, W2, degp)
    p2 = _segsum(y2, srcp, dstp)
    h2, st2 = _combine(p2, y2, degp, b2r)

    return _bnapply(h2, st2, g2r, be2r)
